# baseline (speedup 1.0000x reference)
import functools

import jax
import jax.numpy as jnp
from jax import lax
from jax.experimental import pallas as pl
from jax.experimental.pallas import tpu as pltpu
from jax.experimental.pallas import tpu_sc as plsc

BATCH = 16384
FACTOR = 84
USERS = 1000000
ITEMS = 100000
USTRIDE = 1000064
ISTRIDE = 100096
UMAIN = (USERS // 128) * 128
IMAIN = (ITEMS // 128) * 128
USHIFT = 128 - (USERS - UMAIN)
ISHIFT = 128 - (ITEMS - IMAIN)
NC = 2
NS = 16
NW = NC * NS
B_PER_W = BATCH // NW
G_CHUNK = 128


def _flatten(table_t, tails_t, stride, main):
    f, n = table_t.shape

    def body(in_hbm, tails_hbm, out_hbm, sem):
        cps = []
        for i in range(f):
            cps.append(pltpu.make_async_copy(
                in_hbm.at[i].at[pl.ds(0, main)],
                out_hbm.at[pl.ds(i * stride, main)], sem))
            cps.append(pltpu.make_async_copy(
                tails_hbm.at[i],
                out_hbm.at[pl.ds(i * stride + main, 128)], sem))
        for cp in cps:
            cp.start()
        for cp in cps:
            cp.wait()

    flat = pl.pallas_call(
        body,
        in_specs=[pl.BlockSpec(memory_space=pl.ANY),
                  pl.BlockSpec(memory_space=pl.ANY)],
        out_specs=pl.BlockSpec(memory_space=pl.ANY),
        out_shape=jax.ShapeDtypeStruct((f * stride,), jnp.float32),
        scratch_shapes=[pltpu.SemaphoreType.DMA],
    )(table_t, tails_t)
    return flat.reshape(f, stride)


def _sc_kernel(user_hbm, item_hbm, euf_hbm, eif_hbm, ub_hbm, ib_hbm, b0_hbm,
               out_hbm,
               idx_u, idx_i, idx_ua, idx_ia, ut_v, it_v, ubv, ibv, outv, b0v,
               sem):
    wid = lax.axis_index("s") * NC + lax.axis_index("c")
    base = wid * B_PER_W

    pltpu.sync_copy(user_hbm.at[pl.ds(base, B_PER_W)], idx_u)
    pltpu.sync_copy(item_hbm.at[pl.ds(base, B_PER_W)], idx_i)
    pltpu.sync_copy(b0_hbm, b0v)

    bias_cps = []
    for g in range(B_PER_W // G_CHUNK):
        sl = pl.ds(g * G_CHUNK, G_CHUNK)
        bias_cps.append(pltpu.async_copy(ub_hbm.at[idx_u.at[sl]], ubv.at[sl], sem))
        bias_cps.append(pltpu.async_copy(ib_hbm.at[idx_i.at[sl]], ibv.at[sl], sem))

    def adj_body(g, carry):
        sl = pl.ds(g * 16, 16)
        vu = idx_u[sl]
        idx_ua[sl] = jnp.where(vu >= UMAIN, vu + USHIFT, vu)
        vi = idx_i[sl]
        idx_ia[sl] = jnp.where(vi >= IMAIN, vi + ISHIFT, vi)
        return carry

    lax.fori_loop(0, B_PER_W // 16, adj_body, 0)

    def gather_f(f, carry):
        for g in range(B_PER_W // G_CHUNK):
            sl = pl.ds(g * G_CHUNK, G_CHUNK)
            pltpu.make_async_copy(
                euf_hbm.at[f].at[idx_ua.at[sl]], ut_v.at[f, sl], sem).start()
            pltpu.make_async_copy(
                eif_hbm.at[f].at[idx_ia.at[sl]], it_v.at[f, sl], sem).start()
        return carry

    lax.fori_loop(0, FACTOR, gather_f, 0)

    pltpu.make_async_copy(euf_hbm.at[:, pl.ds(0, B_PER_W)], ut_v, sem).wait()
    pltpu.make_async_copy(eif_hbm.at[:, pl.ds(0, B_PER_W)], it_v, sem).wait()
    for cp in bias_cps:
        cp.wait()

    b0 = b0v[...]

    def grp_body(g, carry):
        sl = pl.ds(g * 16, 16)

        def f_body(f, acc):
            return acc + ut_v[f, sl] * it_v[f, sl]

        acc = lax.fori_loop(0, FACTOR, f_body, jnp.zeros((16,), jnp.float32))
        outv[sl] = acc + ubv[sl] + ibv[sl] + b0
        return carry

    lax.fori_loop(0, B_PER_W // 16, grp_body, 0)

    pltpu.sync_copy(outv, out_hbm.at[pl.ds(base, B_PER_W)])


@jax.jit
def kernel(user, item, embed_user, embed_item, u_bias, i_bias, bias_):
    euf = _flatten(embed_user.T, embed_user[USERS - 128:].T, USTRIDE, UMAIN)
    eif = _flatten(embed_item.T, embed_item[ITEMS - 128:].T, ISTRIDE, IMAIN)

    mesh = plsc.VectorSubcoreMesh(core_axis_name="c", subcore_axis_name="s")
    k = functools.partial(
        pl.kernel,
        mesh=mesh,
        out_type=jax.ShapeDtypeStruct((BATCH,), jnp.float32),
        compiler_params=pltpu.CompilerParams(
            needs_layout_passes=False, use_tc_tiling_on_sc=False),
        scratch_types=[
            pltpu.VMEM((B_PER_W,), jnp.int32),
            pltpu.VMEM((B_PER_W,), jnp.int32),
            pltpu.VMEM((B_PER_W,), jnp.int32),
            pltpu.VMEM((B_PER_W,), jnp.int32),
            pltpu.VMEM((FACTOR, B_PER_W), jnp.float32),
            pltpu.VMEM((FACTOR, B_PER_W), jnp.float32),
            pltpu.VMEM((B_PER_W,), jnp.float32),
            pltpu.VMEM((B_PER_W,), jnp.float32),
            pltpu.VMEM((B_PER_W,), jnp.float32),
            pltpu.VMEM((16,), jnp.float32),
            pltpu.SemaphoreType.DMA,
        ],
    )(_sc_kernel)
    return k(user, item, euf, eif,
             u_bias.reshape(-1), i_bias.reshape(-1),
             jnp.broadcast_to(bias_, (16,)))


# speedup vs baseline: 1.6600x; 1.6600x over previous
import functools

import jax
import jax.numpy as jnp
from jax import lax
from jax.experimental import pallas as pl
from jax.experimental.pallas import tpu as pltpu
from jax.experimental.pallas import tpu_sc as plsc

BATCH = 16384
FACTOR = 84
NC = 2
NS = 16
NW = NC * NS
B_PER_W = BATCH // NW
G_CHUNK = 128


def _sc_kernel(user_hbm, item_hbm, eut_hbm, eit_hbm, ub_hbm, ib_hbm, b0_hbm,
               out_hbm,
               idx_u, idx_i, ut_v, it_v, ubv, ibv, outv, b0v, sem):
    wid = lax.axis_index("s") * NC + lax.axis_index("c")
    base = wid * B_PER_W

    pltpu.sync_copy(user_hbm.at[pl.ds(base, B_PER_W)], idx_u)
    pltpu.sync_copy(item_hbm.at[pl.ds(base, B_PER_W)], idx_i)
    pltpu.sync_copy(b0_hbm, b0v)

    bias_cps = []
    for g in range(B_PER_W // G_CHUNK):
        sl = pl.ds(g * G_CHUNK, G_CHUNK)
        bias_cps.append(pltpu.async_copy(ub_hbm.at[idx_u.at[sl]], ubv.at[sl], sem))
        bias_cps.append(pltpu.async_copy(ib_hbm.at[idx_i.at[sl]], ibv.at[sl], sem))

    def gather_f(f, carry):
        for g in range(B_PER_W // G_CHUNK):
            sl = pl.ds(g * G_CHUNK, G_CHUNK)
            pltpu.make_async_copy(
                eut_hbm.at[f].at[idx_u.at[sl]], ut_v.at[f, sl], sem).start()
            pltpu.make_async_copy(
                eit_hbm.at[f].at[idx_i.at[sl]], it_v.at[f, sl], sem).start()
        return carry

    lax.fori_loop(0, FACTOR, gather_f, 0)

    pltpu.make_async_copy(eut_hbm.at[:, pl.ds(0, B_PER_W)], ut_v, sem).wait()
    pltpu.make_async_copy(eit_hbm.at[:, pl.ds(0, B_PER_W)], it_v, sem).wait()
    for cp in bias_cps:
        cp.wait()

    b0 = b0v[...]

    def grp_body(g, carry):
        sl = pl.ds(g * 16, 16)

        def f_body(f, acc):
            return acc + ut_v[f, sl] * it_v[f, sl]

        acc = lax.fori_loop(0, FACTOR, f_body, jnp.zeros((16,), jnp.float32))
        outv[sl] = acc + ubv[sl] + ibv[sl] + b0
        return carry

    lax.fori_loop(0, B_PER_W // 16, grp_body, 0)

    pltpu.sync_copy(outv, out_hbm.at[pl.ds(base, B_PER_W)])


@jax.jit
def kernel(user, item, embed_user, embed_item, u_bias, i_bias, bias_):
    mesh = plsc.VectorSubcoreMesh(core_axis_name="c", subcore_axis_name="s")
    k = functools.partial(
        pl.kernel,
        mesh=mesh,
        out_type=jax.ShapeDtypeStruct((BATCH,), jnp.float32),
        compiler_params=pltpu.CompilerParams(
            needs_layout_passes=False, use_tc_tiling_on_sc=False),
        scratch_types=[
            pltpu.VMEM((B_PER_W,), jnp.int32),
            pltpu.VMEM((B_PER_W,), jnp.int32),
            pltpu.VMEM((FACTOR, B_PER_W), jnp.float32),
            pltpu.VMEM((FACTOR, B_PER_W), jnp.float32),
            pltpu.VMEM((B_PER_W,), jnp.float32),
            pltpu.VMEM((B_PER_W,), jnp.float32),
            pltpu.VMEM((B_PER_W,), jnp.float32),
            pltpu.VMEM((16,), jnp.float32),
            pltpu.SemaphoreType.DMA,
        ],
    )(_sc_kernel)
    return k(user, item, embed_user.T, embed_item.T,
             u_bias.reshape(-1), i_bias.reshape(-1),
             jnp.broadcast_to(bias_, (16,)))


# speedup vs baseline: 23.8667x; 14.3775x over previous
import functools

import jax
import jax.numpy as jnp
from jax import lax
from jax.experimental import pallas as pl
from jax.experimental.pallas import tpu as pltpu
from jax.experimental.pallas import tpu_sc as plsc

BATCH = 16384
FACTOR = 84
NC = 2
NS = 16
NW = NC * NS
B_PER_W = BATCH // NW
G_CHUNK = 128


def _sc_kernel(user_hbm, item_hbm, eut_hbm, eit_hbm, ub_hbm, ib_hbm, b0_hbm,
               out_hbm,
               idx_u, idx_i, ut_v, it_v, ubv, ibv, outv, b0v, sem):
    wid = lax.axis_index("s") * NC + lax.axis_index("c")
    base = wid * B_PER_W

    pltpu.sync_copy(user_hbm.at[pl.ds(base, B_PER_W)], idx_u)
    pltpu.sync_copy(item_hbm.at[pl.ds(base, B_PER_W)], idx_i)
    pltpu.sync_copy(b0_hbm, b0v)

    bias_cps = []
    for g in range(B_PER_W // G_CHUNK):
        sl = pl.ds(g * G_CHUNK, G_CHUNK)
        bias_cps.append(pltpu.async_copy(ub_hbm.at[idx_u.at[sl]], ubv.at[sl], sem))
        bias_cps.append(pltpu.async_copy(ib_hbm.at[idx_i.at[sl]], ibv.at[sl], sem))

    def gather_f(f, carry):
        for g in range(B_PER_W // G_CHUNK):
            sl = pl.ds(g * G_CHUNK, G_CHUNK)
            pltpu.make_async_copy(
                eut_hbm.at[f].at[idx_u.at[sl]], ut_v.at[f, sl], sem).start()
            pltpu.make_async_copy(
                eit_hbm.at[f].at[idx_i.at[sl]], it_v.at[f, sl], sem).start()
        return carry

    lax.fori_loop(0, FACTOR, gather_f, 0)

    pltpu.make_async_copy(eut_hbm.at[:, pl.ds(0, B_PER_W)], ut_v, sem).wait()
    pltpu.make_async_copy(eit_hbm.at[:, pl.ds(0, B_PER_W)], it_v, sem).wait()
    for cp in bias_cps:
        cp.wait()

    b0 = b0v[...]

    def grp_body(g, carry):
        sl = pl.ds(g * 16, 16)

        def f_body(f, acc):
            return acc + ut_v[f, sl] * it_v[f, sl]

        acc = lax.fori_loop(0, FACTOR, f_body, jnp.zeros((16,), jnp.float32))
        outv[sl] = acc + ubv[sl] + ibv[sl] + b0
        return carry

    lax.fori_loop(0, B_PER_W // 16, grp_body, 0)

    pltpu.sync_copy(outv, out_hbm.at[pl.ds(base, B_PER_W)])


@jax.jit
def kernel(user, item, embed_user, embed_item, u_bias, i_bias, bias_):
    mesh = plsc.VectorSubcoreMesh(core_axis_name="c", subcore_axis_name="s")
    k = functools.partial(
        pl.kernel,
        mesh=mesh,
        out_type=jax.ShapeDtypeStruct((BATCH,), jnp.float32),
        compiler_params=pltpu.CompilerParams(
            needs_layout_passes=False, use_tc_tiling_on_sc=False),
        scratch_types=[
            pltpu.VMEM((B_PER_W,), jnp.int32),
            pltpu.VMEM((B_PER_W,), jnp.int32),
            pltpu.VMEM((FACTOR, B_PER_W), jnp.float32),
            pltpu.VMEM((FACTOR, B_PER_W), jnp.float32),
            pltpu.VMEM((B_PER_W,), jnp.float32),
            pltpu.VMEM((B_PER_W,), jnp.float32),
            pltpu.VMEM((B_PER_W,), jnp.float32),
            pltpu.VMEM((16,), jnp.float32),
            pltpu.SemaphoreType.DMA,
        ],
    )(_sc_kernel)
    from jax.experimental.layout import Format, Layout, with_layout_constraint
    fmt = Layout(major_to_minor=(0, 1), tiling=((8,),))
    eut = with_layout_constraint(embed_user.T, fmt)
    eit = with_layout_constraint(embed_item.T, fmt)
    return k(user, item, eut, eit,
             u_bias.reshape(-1), i_bias.reshape(-1),
             jnp.broadcast_to(bias_, (16,)))
